# baseline (device time: 120453 ns/iter reference)
import jax
import jax.numpy as jnp
from jax import lax
from jax.experimental import pallas as pl
from jax.experimental.pallas import tpu as pltpu

M = 4096
N = 1024
N2 = 2 * N
HALF = M // 2
C = 32
CH = HALF // C


def kernel(x):
    def body(x_ref, out_ref, stage, yrecv, xrecv, stage_sems,
             ysend_sems, yrecv_sems, xsend_sems, xrecv_sems):
        my_x = lax.axis_index("x")
        my_y = lax.axis_index("y")
        ypeer = (my_x, 1 - my_y)
        xpeer = (1 - my_x, my_y)

        barrier_sem = pltpu.get_barrier_semaphore()
        for nbr in (ypeer, xpeer):
            pl.semaphore_signal(
                barrier_sem, inc=1, device_id=nbr,
                device_id_type=pl.DeviceIdType.MESH,
            )
        pl.semaphore_wait(barrier_sem, 2)

        row0 = my_x * HALF
        orow0 = (1 - my_x) * HALF
        my_cols = pl.ds(my_y * N, N)
        peer_cols = pl.ds((1 - my_y) * N, N)

        stages = []
        for c in range(C):
            cp = pltpu.make_async_copy(
                x_ref.at[0, pl.ds(row0 + c * CH, CH), :],
                stage.at[c],
                stage_sems.at[c],
            )
            cp.start()
            stages.append(cp)

        ysends = []
        for c in range(C):
            stages[c].wait()
            rd = pltpu.make_async_remote_copy(
                src_ref=stage.at[c, :, peer_cols],
                dst_ref=yrecv.at[c],
                send_sem=ysend_sems.at[c],
                recv_sem=yrecv_sems.at[c],
                device_id=ypeer,
                device_id_type=pl.DeviceIdType.MESH,
            )
            rd.start()
            ysends.append(rd)
            out_ref[pl.ds(row0 + c * CH, CH), :] = stage[c, :, my_cols]

        xsends = []
        stages2 = []
        for c in range(C):
            ysends[c].wait_recv()
            fwd = pltpu.make_async_remote_copy(
                src_ref=yrecv.at[c],
                dst_ref=xrecv.at[c],
                send_sem=xsend_sems.at[c],
                recv_sem=xrecv_sems.at[c],
                device_id=xpeer,
                device_id_type=pl.DeviceIdType.MESH,
            )
            fwd.start()
            xsends.append(fwd)
            ysends[c].wait_send()
            cp = pltpu.make_async_copy(
                x_ref.at[0, pl.ds(orow0 + c * CH, CH), :],
                stage.at[c],
                stage_sems.at[c],
            )
            cp.start()
            stages2.append(cp)

        out_ref[pl.ds(row0, HALF), :] += yrecv[:, :, :].reshape(HALF, N)

        for c in range(C):
            stages2[c].wait()
            xsends[c].wait_recv()
            out_ref[pl.ds(orow0 + c * CH, CH), :] = (
                stage[c, :, my_cols] + xrecv[c]
            )

        for c in range(C):
            xsends[c].wait_send()

    return pl.pallas_call(
        body,
        out_shape=jax.ShapeDtypeStruct((M, N), jnp.float32),
        in_specs=[pl.BlockSpec(memory_space=pl.ANY)],
        out_specs=pl.BlockSpec(memory_space=pltpu.VMEM),
        scratch_shapes=[
            pltpu.VMEM((C, CH, N2), jnp.float32),
            pltpu.VMEM((C, CH, N), jnp.float32),
            pltpu.VMEM((C, CH, N), jnp.float32),
            pltpu.SemaphoreType.DMA((C,)),
            pltpu.SemaphoreType.DMA((C,)),
            pltpu.SemaphoreType.DMA((C,)),
            pltpu.SemaphoreType.DMA((C,)),
            pltpu.SemaphoreType.DMA((C,)),
        ],
        compiler_params=pltpu.CompilerParams(
            collective_id=0,
            vmem_limit_bytes=100 * 1024 * 1024,
        ),
    )(x)
